# Initial kernel scaffold; baseline (speedup 1.0000x reference)
import jax
import jax.numpy as jnp
from jax.experimental import pallas as pl


def kernel(x, gate_w, w1, w2, w3):
    raise NotImplementedError("write your pallas kernel here")


# speedup vs baseline: 1.8791x; 1.8791x over previous
import jax
import jax.numpy as jnp
from jax.experimental import pallas as pl
from jax.experimental.pallas import tpu as pltpu

N_EXPERT = 8
D_MODEL = 1024
D_FF = 1024
T_TOK = 2048


def _nt_dot(a, b):
    return jax.lax.dot_general(a, b, (((1,), (1,)), ((), ())),
                               preferred_element_type=jnp.float32)


def _moe_body(x_ref, gw_ref, w1_ref, w2_ref, w3_ref, o_ref):
    e = pl.program_id(0)
    x = x_ref[...]
    gw = gw_ref[...]
    router = _nt_dot(x, gw)
    iota8 = jax.lax.broadcasted_iota(jnp.int32, router.shape, 1)
    m0 = jnp.max(router, axis=1, keepdims=True)
    i0 = jnp.min(jnp.where(router == m0, iota8, N_EXPERT), axis=1, keepdims=True)
    masked = jnp.where(iota8 == i0, -jnp.inf, router)
    m1 = jnp.max(masked, axis=1, keepdims=True)
    i1 = jnp.min(jnp.where(masked == m1, iota8, N_EXPERT), axis=1, keepdims=True)
    e1 = jnp.exp(m1 - m0)
    denom = 1.0 + e1
    p0 = 1.0 / denom
    p1 = e1 / denom
    we = p0 * (i0 == e).astype(jnp.float32) + p1 * (i1 == e).astype(jnp.float32)

    w1 = w1_ref[0]
    w2 = w2_ref[0]
    w3 = w3_ref[0]
    h1 = _nt_dot(x, w1)
    h2 = _nt_dot(x, w2)
    h = (h1 * (1.0 / (1.0 + jnp.exp(-h1)))) * h2
    out = _nt_dot(h, w3)

    @pl.when(e == 0)
    def _init():
        o_ref[...] = jnp.zeros_like(o_ref)

    o_ref[...] += we * out


def kernel(x, gate_w, w1, w2, w3):
    Bq, Tq, C = x.shape
    xf = x.reshape(Tq, C)
    out = pl.pallas_call(
        _moe_body,
        grid=(N_EXPERT,),
        in_specs=[
            pl.BlockSpec((T_TOK, D_MODEL), lambda e: (0, 0)),
            pl.BlockSpec((N_EXPERT, D_MODEL), lambda e: (0, 0)),
            pl.BlockSpec((1, D_FF, D_MODEL), lambda e: (e, 0, 0)),
            pl.BlockSpec((1, D_FF, D_MODEL), lambda e: (e, 0, 0)),
            pl.BlockSpec((1, D_MODEL, D_FF), lambda e: (e, 0, 0)),
        ],
        out_specs=pl.BlockSpec((T_TOK, D_MODEL), lambda e: (0, 0)),
        out_shape=jax.ShapeDtypeStruct((T_TOK, D_MODEL), jnp.float32),
        compiler_params=pltpu.CompilerParams(
            dimension_semantics=("arbitrary",),
        ),
    )(xf, gate_w, w1, w2, w3)
    return out.reshape(Bq, Tq, C)
